# Initial kernel scaffold; baseline (speedup 1.0000x reference)
import jax
import jax.numpy as jnp
from jax.experimental import pallas as pl


def kernel(x_enc, edge_index, W1, b1, W2, b2):
    raise NotImplementedError("write your pallas kernel here")


# speedup vs baseline: 9.5189x; 9.5189x over previous
import functools

import jax
import jax.numpy as jnp
from jax import lax
from jax.experimental import pallas as pl
from jax.experimental.pallas import tpu as pltpu
from jax.experimental.pallas import tpu_sc as plsc

N = 10000
NP = 10240
D = 128
E = 320000
EP = 327680
CHUNK = 128
CPT = EP // 32 // CHUNK
RPT = NP // 16
ROWBLK = 1024
GRID = NP // ROWBLK


def _sc_degree_body(dst_hbm, out_hbm, didx, buf, deg):
    c = lax.axis_index("c")
    s = lax.axis_index("s")
    wid = s * 2 + c

    def fill(val):
        def go(i, _):
            for k in range(D // 16):
                buf[i, pl.ds(k * 16, 16)] = val
            return 0
        lax.fori_loop(0, CHUNK, go, 0)

    fill(jnp.zeros((16,), jnp.float32))
    for t in range(RPT // CHUNK):
        pltpu.sync_copy(buf, deg.at[pl.ds(s * RPT + t * CHUNK, CHUNK)])
    fill(jnp.ones((16,), jnp.float32))
    pltpu.sync_copy(dst_hbm.at[pl.ds(wid * CPT, CPT)], didx)
    plsc.subcore_barrier()

    def body(j, _):
        pltpu.sync_copy(buf, deg.at[didx.at[j]], add=True)
        return 0

    lax.fori_loop(0, CPT, body, 0)
    plsc.subcore_barrier()
    pltpu.sync_copy(deg.at[pl.ds(s * RPT, RPT)],
                    out_hbm.at[pl.ds(c * NP + s * RPT, RPT)])


def _sc_edge_accum_body(h2_hbm, src_hbm, dst_hbm, out_hbm, sidx, didx, rows, acc):
    c = lax.axis_index("c")
    s = lax.axis_index("s")
    wid = s * 2 + c

    def zrow(i, _):
        for k in range(D // 16):
            rows[i, pl.ds(k * 16, 16)] = jnp.zeros((16,), jnp.float32)
        return 0

    lax.fori_loop(0, CHUNK, zrow, 0)
    for t in range(RPT // CHUNK):
        pltpu.sync_copy(rows, acc.at[pl.ds(s * RPT + t * CHUNK, CHUNK)])
    pltpu.sync_copy(src_hbm.at[pl.ds(wid * CPT, CPT)], sidx)
    pltpu.sync_copy(dst_hbm.at[pl.ds(wid * CPT, CPT)], didx)
    plsc.subcore_barrier()

    def body(j, _):
        pltpu.sync_copy(h2_hbm.at[sidx.at[j]], rows)
        pltpu.sync_copy(rows, acc.at[didx.at[j]], add=True)
        return 0

    lax.fori_loop(0, CPT, body, 0)
    plsc.subcore_barrier()
    pltpu.sync_copy(acc.at[pl.ds(s * RPT, RPT)],
                    out_hbm.at[pl.ds(c * NP + s * RPT, RPT)])


@functools.lru_cache(maxsize=None)
def _sc_kernels():
    mesh = plsc.VectorSubcoreMesh(
        core_axis_name="c", subcore_axis_name="s", num_cores=2, num_subcores=16)
    sc_degree = pl.kernel(
        _sc_degree_body,
        out_type=jax.ShapeDtypeStruct((2 * NP, D), jnp.float32),
        mesh=mesh,
        scratch_types=[
            pltpu.VMEM((CPT, CHUNK), jnp.int32),
            pltpu.VMEM((CHUNK, D), jnp.float32),
            pltpu.VMEM_SHARED((NP, D), jnp.float32),
        ],
    )
    sc_edge_accum = pl.kernel(
        _sc_edge_accum_body,
        out_type=jax.ShapeDtypeStruct((2 * NP, D), jnp.float32),
        mesh=mesh,
        scratch_types=[
            pltpu.VMEM((CPT, CHUNK), jnp.int32),
            pltpu.VMEM((CPT, CHUNK), jnp.int32),
            pltpu.VMEM((CHUNK, D), jnp.float32),
            pltpu.VMEM_SHARED((NP, D), jnp.float32),
        ],
    )
    return sc_degree, sc_edge_accum


def _dinv_block(deg_ref):
    deg = deg_ref[0, :, 0:1] + deg_ref[1, :, 0:1] + 1.0
    return lax.rsqrt(jnp.maximum(deg, 1e-12))


def _tc1_body(x_ref, w_ref, deg_ref, out_ref):
    dinv = _dinv_block(deg_ref)
    out_ref[...] = jnp.dot(x_ref[...], w_ref[...],
                           preferred_element_type=jnp.float32,
                           precision=lax.Precision.HIGHEST) * dinv


def _tc2_body(acc_ref, h2_ref, deg_ref, b_ref, w_ref, out_ref):
    dinv = _dinv_block(deg_ref)
    pre = acc_ref[0] + acc_ref[1] + h2_ref[...]
    x2 = jnp.maximum(pre * dinv + b_ref[...], 0.0)
    out_ref[...] = jnp.dot(x2, w_ref[...],
                           preferred_element_type=jnp.float32,
                           precision=lax.Precision.HIGHEST) * dinv


def _tc3_body(acc_ref, h2_ref, deg_ref, b_ref, out_ref):
    dinv = _dinv_block(deg_ref)
    pre = acc_ref[0] + acc_ref[1] + h2_ref[...]
    out_ref[...] = jnp.maximum(pre * dinv + b_ref[...], 0.0)


_spec_rows = pl.BlockSpec((ROWBLK, D), lambda i: (i, 0))
_spec_acc = pl.BlockSpec((2, ROWBLK, D), lambda i: (0, i, 0))
_spec_deg = pl.BlockSpec((2, ROWBLK, D), lambda i: (0, i, 0))
_spec_w = pl.BlockSpec((D, D), lambda i: (0, 0))
_spec_b = pl.BlockSpec((1, D), lambda i: (0, 0))
_out_rows = jax.ShapeDtypeStruct((NP, D), jnp.float32)

_tc1 = pl.pallas_call(
    _tc1_body, grid=(GRID,),
    in_specs=[_spec_rows, _spec_w, _spec_deg],
    out_specs=_spec_rows, out_shape=_out_rows)

_tc2 = pl.pallas_call(
    _tc2_body, grid=(GRID,),
    in_specs=[_spec_acc, _spec_rows, _spec_deg, _spec_b, _spec_w],
    out_specs=_spec_rows, out_shape=_out_rows)

_tc3 = pl.pallas_call(
    _tc3_body, grid=(GRID,),
    in_specs=[_spec_acc, _spec_rows, _spec_deg, _spec_b],
    out_specs=_spec_rows, out_shape=_out_rows)


def kernel(x_enc, edge_index, W1, b1, W2, b2):
    src = edge_index[0].astype(jnp.int32)
    dst = edge_index[1].astype(jnp.int32)
    npad = EP - E
    pad_src = jnp.full((npad,), N, dtype=jnp.int32)
    pad_dst = N + (jnp.arange(npad, dtype=jnp.int32) % (NP - N))
    src2d = jnp.concatenate([src, pad_src]).reshape(EP // CHUNK, CHUNK)
    dst2d = jnp.concatenate([dst, pad_dst]).reshape(EP // CHUNK, CHUNK)
    x_pad = jnp.zeros((NP, D), jnp.float32).at[:N].set(x_enc)
    b1r = b1.reshape(1, D)
    b2r = b2.reshape(1, D)

    sc_degree, sc_edge_accum = _sc_kernels()
    degp = sc_degree(dst2d).reshape(2, NP, D)
    h2_1 = _tc1(x_pad, W1, degp)
    acc1 = sc_edge_accum(h2_1, src2d, dst2d).reshape(2, NP, D)
    h2_2 = _tc2(acc1, h2_1, degp, b1r, W2)
    acc2 = sc_edge_accum(h2_2, src2d, dst2d).reshape(2, NP, D)
    out = _tc3(acc2, h2_2, degp, b2r)
    return out[:N]


# speedup vs baseline: 9.7513x; 1.0244x over previous
import functools

import jax
import jax.numpy as jnp
from jax import lax
from jax.experimental import pallas as pl
from jax.experimental.pallas import tpu as pltpu
from jax.experimental.pallas import tpu_sc as plsc

N = 10000
NP = 10240
D = 128
E = 320000
EP = 327680
CHUNK = 128
CPT = EP // 32 // CHUNK
RPT = NP // 16
ROWBLK = 1024
GRID = NP // ROWBLK


def _sc_degree_body(dst_hbm, out_hbm, didx, buf, deg):
    c = lax.axis_index("c")
    s = lax.axis_index("s")
    wid = s * 2 + c

    def fill(val):
        def go(i, _):
            for k in range(D // 16):
                buf[i, pl.ds(k * 16, 16)] = val
            return 0
        lax.fori_loop(0, CHUNK, go, 0)

    fill(jnp.zeros((16,), jnp.float32))
    for t in range(RPT // CHUNK):
        pltpu.sync_copy(buf, deg.at[pl.ds(s * RPT + t * CHUNK, CHUNK)])
    fill(jnp.ones((16,), jnp.float32))
    pltpu.sync_copy(dst_hbm.at[pl.ds(wid * CPT, CPT)], didx)
    plsc.subcore_barrier()

    def body(j, _):
        pltpu.sync_copy(buf, deg.at[didx.at[j]], add=True)
        return 0

    lax.fori_loop(0, CPT, body, 0)
    plsc.subcore_barrier()
    pltpu.sync_copy(deg.at[pl.ds(s * RPT, RPT)],
                    out_hbm.at[pl.ds(c * NP + s * RPT, RPT)])


NQ = 5
QC = CPT // NQ


def _sc_edge_accum_body(h2_hbm, src_hbm, dst_hbm, out_hbm,
                        i0s, i0d, i1s, i1d, r0, r1, acc, g0, g1):
    islot = ((i0s, i0d), (i1s, i1d))
    c = lax.axis_index("c")
    s = lax.axis_index("s")
    wid = s * 2 + c
    base = wid * CPT

    def load_idx(q):
        ss, dd = islot[q % 2]
        pltpu.sync_copy(src_hbm.at[pl.ds(base + q * QC, QC)], ss)
        pltpu.sync_copy(dst_hbm.at[pl.ds(base + q * QC, QC)], dd)

    def zrow(i, _):
        for k in range(D // 16):
            r0[i, pl.ds(k * 16, 16)] = jnp.zeros((16,), jnp.float32)
        return 0

    lax.fori_loop(0, CHUNK, zrow, 0)
    for t in range(RPT // CHUNK):
        pltpu.sync_copy(r0, acc.at[pl.ds(s * RPT + t * CHUNK, CHUNK)])
    plsc.subcore_barrier()

    for q in range(NQ):
        ss, dd = islot[q % 2]
        load_idx(q)

        def pair(j, _):
            h0 = pltpu.async_copy(h2_hbm.at[ss.at[2 * j]], r0, g0)
            h1 = pltpu.async_copy(h2_hbm.at[ss.at[2 * j + 1]], r1, g1)
            h0.wait()
            pltpu.sync_copy(r0, acc.at[dd.at[2 * j]], add=True)
            h1.wait()
            pltpu.sync_copy(r1, acc.at[dd.at[2 * j + 1]], add=True)
            return 0

        lax.fori_loop(0, QC // 2, pair, 0)

    plsc.subcore_barrier()
    pltpu.sync_copy(acc.at[pl.ds(s * RPT, RPT)],
                    out_hbm.at[pl.ds(c * NP + s * RPT, RPT)])


@functools.lru_cache(maxsize=None)
def _sc_kernels():
    mesh = plsc.VectorSubcoreMesh(
        core_axis_name="c", subcore_axis_name="s", num_cores=2, num_subcores=16)
    sc_degree = pl.kernel(
        _sc_degree_body,
        out_type=jax.ShapeDtypeStruct((2 * NP, D), jnp.float32),
        mesh=mesh,
        scratch_types=[
            pltpu.VMEM((CPT, CHUNK), jnp.int32),
            pltpu.VMEM((CHUNK, D), jnp.float32),
            pltpu.VMEM_SHARED((NP, D), jnp.float32),
        ],
    )
    sc_edge_accum = pl.kernel(
        _sc_edge_accum_body,
        out_type=jax.ShapeDtypeStruct((2 * NP, D), jnp.float32),
        mesh=mesh,
        scratch_types=[pltpu.VMEM((QC, CHUNK), jnp.int32)] * 4 +
        [pltpu.VMEM((CHUNK, D), jnp.float32)] * 2 + [
            pltpu.VMEM_SHARED((NP, D), jnp.float32),
        ] + [pltpu.SemaphoreType.DMA] * 2,
    )
    return sc_degree, sc_edge_accum


def _dinv_block(deg_ref):
    deg = deg_ref[0, :, 0:1] + deg_ref[1, :, 0:1] + 1.0
    return lax.rsqrt(jnp.maximum(deg, 1e-12))


def _tc1_body(x_ref, w_ref, deg_ref, out_ref):
    dinv = _dinv_block(deg_ref)
    out_ref[...] = jnp.dot(x_ref[...], w_ref[...],
                           preferred_element_type=jnp.float32,
                           precision=lax.Precision.HIGHEST) * dinv


def _tc2_body(acc_ref, h2_ref, deg_ref, b_ref, w_ref, out_ref):
    dinv = _dinv_block(deg_ref)
    pre = acc_ref[0] + acc_ref[1] + h2_ref[...]
    x2 = jnp.maximum(pre * dinv + b_ref[...], 0.0)
    out_ref[...] = jnp.dot(x2, w_ref[...],
                           preferred_element_type=jnp.float32,
                           precision=lax.Precision.HIGHEST) * dinv


def _tc3_body(acc_ref, h2_ref, deg_ref, b_ref, out_ref):
    dinv = _dinv_block(deg_ref)
    pre = acc_ref[0] + acc_ref[1] + h2_ref[...]
    out_ref[...] = jnp.maximum(pre * dinv + b_ref[...], 0.0)


_spec_rows = pl.BlockSpec((ROWBLK, D), lambda i: (i, 0))
_spec_acc = pl.BlockSpec((2, ROWBLK, D), lambda i: (0, i, 0))
_spec_deg = pl.BlockSpec((2, ROWBLK, D), lambda i: (0, i, 0))
_spec_w = pl.BlockSpec((D, D), lambda i: (0, 0))
_spec_b = pl.BlockSpec((1, D), lambda i: (0, 0))
_out_rows = jax.ShapeDtypeStruct((NP, D), jnp.float32)

_tc1 = pl.pallas_call(
    _tc1_body, grid=(GRID,),
    in_specs=[_spec_rows, _spec_w, _spec_deg],
    out_specs=_spec_rows, out_shape=_out_rows)

_tc2 = pl.pallas_call(
    _tc2_body, grid=(GRID,),
    in_specs=[_spec_acc, _spec_rows, _spec_deg, _spec_b, _spec_w],
    out_specs=_spec_rows, out_shape=_out_rows)

_tc3 = pl.pallas_call(
    _tc3_body, grid=(GRID,),
    in_specs=[_spec_acc, _spec_rows, _spec_deg, _spec_b],
    out_specs=_spec_rows, out_shape=_out_rows)


def kernel(x_enc, edge_index, W1, b1, W2, b2):
    src = edge_index[0].astype(jnp.int32)
    dst = edge_index[1].astype(jnp.int32)
    npad = EP - E
    pad_src = jnp.full((npad,), N, dtype=jnp.int32)
    pad_dst = N + (jnp.arange(npad, dtype=jnp.int32) % (NP - N))
    src2d = jnp.concatenate([src, pad_src]).reshape(EP // CHUNK, CHUNK)
    dst2d = jnp.concatenate([dst, pad_dst]).reshape(EP // CHUNK, CHUNK)
    x_pad = jnp.zeros((NP, D), jnp.float32).at[:N].set(x_enc)
    b1r = b1.reshape(1, D)
    b2r = b2.reshape(1, D)

    sc_degree, sc_edge_accum = _sc_kernels()
    degp = sc_degree(dst2d).reshape(2, NP, D)
    h2_1 = _tc1(x_pad, W1, degp)
    acc1 = sc_edge_accum(h2_1, src2d, dst2d).reshape(2, NP, D)
    h2_2 = _tc2(acc1, h2_1, degp, b1r, W2)
    acc2 = sc_edge_accum(h2_2, src2d, dst2d).reshape(2, NP, D)
    out = _tc3(acc2, h2_2, degp, b2r)
    return out[:N]


# speedup vs baseline: 10.2835x; 1.0546x over previous
import functools

import jax
import jax.numpy as jnp
from jax import lax
from jax.experimental import pallas as pl
from jax.experimental.pallas import tpu as pltpu
from jax.experimental.pallas import tpu_sc as plsc

N = 10000
NP = 10240
D = 128
E = 320000
EP = 327680
CHUNK = 128
CPT = EP // 32 // CHUNK
RPT = NP // 16
ROWBLK = 1024
GRID = NP // ROWBLK


def _sc_degree_body(dst_hbm, out_hbm, didx, buf, deg):
    c = lax.axis_index("c")
    s = lax.axis_index("s")
    wid = s * 2 + c

    def fill(val):
        def go(i, _):
            for k in range(D // 16):
                buf[i, pl.ds(k * 16, 16)] = val
            return 0
        lax.fori_loop(0, CHUNK, go, 0)

    fill(jnp.zeros((16,), jnp.float32))
    for t in range(RPT // CHUNK):
        pltpu.sync_copy(buf, deg.at[pl.ds(s * RPT + t * CHUNK, CHUNK)])
    fill(jnp.ones((16,), jnp.float32))
    pltpu.sync_copy(dst_hbm.at[pl.ds(wid * CPT, CPT)], didx)
    plsc.subcore_barrier()

    def body(j, _):
        pltpu.sync_copy(buf, deg.at[didx.at[j]], add=True)
        return 0

    lax.fori_loop(0, CPT, body, 0)
    plsc.subcore_barrier()
    pltpu.sync_copy(deg.at[pl.ds(s * RPT, RPT)],
                    out_hbm.at[pl.ds(c * NP + s * RPT, RPT)])


NQ = 5
QC = CPT // NQ


def _sc_edge_accum_body(h2_hbm, src_hbm, dst_hbm, out_hbm,
                        i0s, i0d, i1s, i1d, r0, r1, acc, g0, g1, s0, s1):
    islot = ((i0s, i0d), (i1s, i1d))
    c = lax.axis_index("c")
    s = lax.axis_index("s")
    wid = s * 2 + c
    base = wid * CPT

    def load_idx(q):
        ss, dd = islot[q % 2]
        pltpu.sync_copy(src_hbm.at[pl.ds(base + q * QC, QC)], ss)
        pltpu.sync_copy(dst_hbm.at[pl.ds(base + q * QC, QC)], dd)

    def zrow(i, _):
        for k in range(D // 16):
            r0[i, pl.ds(k * 16, 16)] = jnp.zeros((16,), jnp.float32)
        return 0

    lax.fori_loop(0, CHUNK, zrow, 0)
    for t in range(RPT // CHUNK):
        pltpu.sync_copy(r0, acc.at[pl.ds(s * RPT + t * CHUNK, CHUNK)])
    plsc.subcore_barrier()

    rows = (r0, r1)
    gsem = (g0, g1)
    ssem = (s0, s1)

    for q in range(NQ):
        ss, dd = islot[q % 2]
        load_idx(q)

        def start_gather(e, b):
            pltpu.async_copy(h2_hbm.at[ss.at[e]], rows[b], gsem[b])

        def wait_gather(e, b):
            pltpu.make_async_copy(h2_hbm.at[ss.at[e]], rows[b],
                                  gsem[b]).wait()

        def start_scatter(e, b):
            pltpu.async_copy(rows[b], acc.at[dd.at[e]], ssem[b], add=True)

        def wait_scatter(e, b):
            pltpu.make_async_copy(rows[b], acc.at[dd.at[e]], ssem[b]).wait()

        for b in range(2):
            start_gather(b, b)

        def pair(j, _):
            for b in range(2):
                wait_gather(2 * j + b, b)
                start_scatter(2 * j + b, b)
            for b in range(2):
                wait_scatter(2 * j + b, b)
                start_gather(2 * j + 2 + b, b)
            return 0

        lax.fori_loop(0, QC // 2 - 1, pair, 0)
        for b in range(2):
            wait_gather(QC - 2 + b, b)
            start_scatter(QC - 2 + b, b)
        for b in range(2):
            wait_scatter(QC - 2 + b, b)

    plsc.subcore_barrier()
    pltpu.sync_copy(acc.at[pl.ds(s * RPT, RPT)],
                    out_hbm.at[pl.ds(c * NP + s * RPT, RPT)])


@functools.lru_cache(maxsize=None)
def _sc_kernels():
    mesh = plsc.VectorSubcoreMesh(
        core_axis_name="c", subcore_axis_name="s", num_cores=2, num_subcores=16)
    sc_degree = pl.kernel(
        _sc_degree_body,
        out_type=jax.ShapeDtypeStruct((2 * NP, D), jnp.float32),
        mesh=mesh,
        scratch_types=[
            pltpu.VMEM((CPT, CHUNK), jnp.int32),
            pltpu.VMEM((CHUNK, D), jnp.float32),
            pltpu.VMEM_SHARED((NP, D), jnp.float32),
        ],
    )
    sc_edge_accum = pl.kernel(
        _sc_edge_accum_body,
        out_type=jax.ShapeDtypeStruct((2 * NP, D), jnp.float32),
        mesh=mesh,
        scratch_types=[pltpu.VMEM((QC, CHUNK), jnp.int32)] * 4 +
        [pltpu.VMEM((CHUNK, D), jnp.float32)] * 2 + [
            pltpu.VMEM_SHARED((NP, D), jnp.float32),
        ] + [pltpu.SemaphoreType.DMA] * 4,
    )
    return sc_degree, sc_edge_accum


def _dinv_block(deg_ref):
    deg = deg_ref[0, :, 0:1] + deg_ref[1, :, 0:1] + 1.0
    return lax.rsqrt(jnp.maximum(deg, 1e-12))


def _tc1_body(x_ref, w_ref, deg_ref, out_ref):
    dinv = _dinv_block(deg_ref)
    out_ref[...] = jnp.dot(x_ref[...], w_ref[...],
                           preferred_element_type=jnp.float32,
                           precision=lax.Precision.HIGHEST) * dinv


def _tc2_body(acc_ref, h2_ref, deg_ref, b_ref, w_ref, out_ref):
    dinv = _dinv_block(deg_ref)
    pre = acc_ref[0] + acc_ref[1] + h2_ref[...]
    x2 = jnp.maximum(pre * dinv + b_ref[...], 0.0)
    out_ref[...] = jnp.dot(x2, w_ref[...],
                           preferred_element_type=jnp.float32,
                           precision=lax.Precision.HIGHEST) * dinv


def _tc3_body(acc_ref, h2_ref, deg_ref, b_ref, out_ref):
    dinv = _dinv_block(deg_ref)
    pre = acc_ref[0] + acc_ref[1] + h2_ref[...]
    out_ref[...] = jnp.maximum(pre * dinv + b_ref[...], 0.0)


_spec_rows = pl.BlockSpec((ROWBLK, D), lambda i: (i, 0))
_spec_acc = pl.BlockSpec((2, ROWBLK, D), lambda i: (0, i, 0))
_spec_deg = pl.BlockSpec((2, ROWBLK, D), lambda i: (0, i, 0))
_spec_w = pl.BlockSpec((D, D), lambda i: (0, 0))
_spec_b = pl.BlockSpec((1, D), lambda i: (0, 0))
_out_rows = jax.ShapeDtypeStruct((NP, D), jnp.float32)

_tc1 = pl.pallas_call(
    _tc1_body, grid=(GRID,),
    in_specs=[_spec_rows, _spec_w, _spec_deg],
    out_specs=_spec_rows, out_shape=_out_rows)

_tc2 = pl.pallas_call(
    _tc2_body, grid=(GRID,),
    in_specs=[_spec_acc, _spec_rows, _spec_deg, _spec_b, _spec_w],
    out_specs=_spec_rows, out_shape=_out_rows)

_tc3 = pl.pallas_call(
    _tc3_body, grid=(GRID,),
    in_specs=[_spec_acc, _spec_rows, _spec_deg, _spec_b],
    out_specs=_spec_rows, out_shape=_out_rows)


def kernel(x_enc, edge_index, W1, b1, W2, b2):
    src = edge_index[0].astype(jnp.int32)
    dst = edge_index[1].astype(jnp.int32)
    npad = EP - E
    pad_src = jnp.full((npad,), N, dtype=jnp.int32)
    pad_dst = N + (jnp.arange(npad, dtype=jnp.int32) % (NP - N))
    src2d = jnp.concatenate([src, pad_src]).reshape(EP // CHUNK, CHUNK)
    dst2d = jnp.concatenate([dst, pad_dst]).reshape(EP // CHUNK, CHUNK)
    x_pad = jnp.zeros((NP, D), jnp.float32).at[:N].set(x_enc)
    b1r = b1.reshape(1, D)
    b2r = b2.reshape(1, D)

    sc_degree, sc_edge_accum = _sc_kernels()
    degp = sc_degree(dst2d).reshape(2, NP, D)
    h2_1 = _tc1(x_pad, W1, degp)
    acc1 = sc_edge_accum(h2_1, src2d, dst2d).reshape(2, NP, D)
    h2_2 = _tc2(acc1, h2_1, degp, b1r, W2)
    acc2 = sc_edge_accum(h2_2, src2d, dst2d).reshape(2, NP, D)
    out = _tc3(acc2, h2_2, degp, b2r)
    return out[:N]
